# Initial kernel scaffold; baseline (speedup 1.0000x reference)
import jax
import jax.numpy as jnp
from jax.experimental import pallas as pl


def kernel(numbers, bias, lin_w, cat_params, categories, embed_idx):
    raise NotImplementedError("write your pallas kernel here")


# speedup vs baseline: 1.0283x; 1.0283x over previous
import functools

import jax
import jax.numpy as jnp
from jax import lax
from jax.experimental import pallas as pl
from jax.experimental.pallas import tpu as pltpu
from jax.experimental.pallas import tpu_sc as plsc

_B = 16384
_F = 26
_ND = 13
_NC = 2
_NS = 16
_NW = _NC * _NS
_NB = _B // _NW
_NIDX = _F * _NB
_L = 16


def _sc_body(cat_hbm, off_hbm, num_hbm, cst_hbm, table_hbm, out_hbm,
             idx_v, off_v, val_v, num_v, cst_v, out_v, gsem):
    wid = lax.axis_index("s") * _NC + lax.axis_index("c")
    base = wid * _NB

    pltpu.sync_copy(cat_hbm.at[wid], idx_v)
    pltpu.sync_copy(off_hbm, off_v)
    pltpu.sync_copy(num_hbm.at[wid], num_v)
    pltpu.sync_copy(cst_hbm, cst_v)

    def add_off(i, _):
        s = pl.ds(i * _L, _L)
        idx_v[s] = idx_v[s] + off_v[s]
        return 0

    lax.fori_loop(0, _NIDX // _L, add_off, 0)

    gather = pltpu.async_copy(table_hbm.at[idx_v], val_v, gsem)

    def dense(c, _):
        s = pl.ds(c * _L, _L)
        acc = cst_v[pl.ds(_ND * _L, _L)]
        for j in range(_ND):
            acc = acc + num_v[pl.ds(j * _NB + c * _L, _L)] * cst_v[pl.ds(j * _L, _L)]
        out_v[s] = acc
        return 0

    lax.fori_loop(0, _NB // _L, dense, 0)

    gather.wait()

    def reduce(c, _):
        s = pl.ds(c * _L, _L)
        acc = out_v[s]
        for f in range(_F):
            acc = acc + val_v[pl.ds(f * _NB + c * _L, _L)]
        out_v[s] = acc
        return 0

    lax.fori_loop(0, _NB // _L, reduce, 0)

    pltpu.sync_copy(out_v, out_hbm.at[pl.ds(base, _NB)])


@jax.jit
def _cat_linear_sc(cat_prep, off_flat, num_prep, cst, table_flat):
    mesh = plsc.VectorSubcoreMesh(core_axis_name="c", subcore_axis_name="s")
    k = pl.kernel(
        _sc_body,
        out_type=jax.ShapeDtypeStruct((_B,), jnp.float32),
        mesh=mesh,
        scratch_types=[
            pltpu.VMEM((_NIDX,), jnp.int32),
            pltpu.VMEM((_NIDX,), jnp.int32),
            pltpu.VMEM((_NIDX,), jnp.float32),
            pltpu.VMEM((_ND * _NB,), jnp.float32),
            pltpu.VMEM(((_ND + 1) * _L,), jnp.float32),
            pltpu.VMEM((_NB,), jnp.float32),
            pltpu.SemaphoreType.DMA,
        ],
    )
    return k(cat_prep, off_flat, num_prep, cst, table_flat)


def kernel(numbers, bias, lin_w, cat_params, categories, embed_idx):
    cat_prep = (
        categories.reshape(_NW, _NB, _F).transpose(0, 2, 1).reshape(_NW, _NIDX)
    )
    off_flat = jnp.repeat(embed_idx.astype(jnp.int32), _NB)
    num_prep = (
        numbers.reshape(_NW, _NB, _ND).transpose(0, 2, 1).reshape(_NW, _ND * _NB)
    )
    cst = jnp.concatenate(
        [jnp.repeat(lin_w.reshape(_ND), _L), jnp.repeat(bias.reshape(1), _L)]
    )
    table_flat = cat_params.reshape(-1)
    out = _cat_linear_sc(cat_prep, off_flat, num_prep, cst, table_flat)
    return out.reshape(_B, 1)


# speedup vs baseline: 1.0293x; 1.0010x over previous
import functools

import jax
import jax.numpy as jnp
from jax import lax
from jax.experimental import pallas as pl
from jax.experimental.pallas import tpu as pltpu
from jax.experimental.pallas import tpu_sc as plsc

_B = 16384
_F = 26
_ND = 13
_NC = 2
_NS = 16
_NW = _NC * _NS
_NB = _B // _NW
_NIDX = _F * _NB
_L = 16
_NSTREAM = 4


def _sc_body(cat_hbm, off_hbm, num_hbm, cst_hbm, table_hbm, out_hbm,
             idx_v, off_v, val_v, num_v, cst_v, out_v, gsem):
    wid = lax.axis_index("s") * _NC + lax.axis_index("c")
    base = wid * _NB

    pltpu.sync_copy(cat_hbm.at[wid], idx_v)
    pltpu.sync_copy(off_hbm, off_v)
    pltpu.sync_copy(num_hbm.at[wid], num_v)
    pltpu.sync_copy(cst_hbm, cst_v)

    def add_off(i, _):
        s = pl.ds(i * _L, _L)
        idx_v[s] = idx_v[s] + off_v[s]
        return 0

    lax.fori_loop(0, _NIDX // _L, add_off, 0)

    chunk = _NIDX // _NSTREAM
    gathers = [
        pltpu.async_copy(
            table_hbm.at[idx_v.at[pl.ds(q * chunk, chunk)]],
            val_v.at[pl.ds(q * chunk, chunk)],
            gsem.at[q],
        )
        for q in range(_NSTREAM)
    ]

    def dense(c, _):
        s = pl.ds(c * _L, _L)
        acc = cst_v[pl.ds(_ND * _L, _L)]
        for j in range(_ND):
            acc = acc + num_v[pl.ds(j * _NB + c * _L, _L)] * cst_v[pl.ds(j * _L, _L)]
        out_v[s] = acc
        return 0

    lax.fori_loop(0, _NB // _L, dense, 0)

    for g in gathers:
        g.wait()

    def reduce(c, _):
        s = pl.ds(c * _L, _L)
        acc = out_v[s]
        for f in range(_F):
            acc = acc + val_v[pl.ds(f * _NB + c * _L, _L)]
        out_v[s] = acc
        return 0

    lax.fori_loop(0, _NB // _L, reduce, 0)

    pltpu.sync_copy(out_v, out_hbm.at[pl.ds(base, _NB)])


@jax.jit
def _cat_linear_sc(cat_prep, off_flat, num_prep, cst, table_flat):
    mesh = plsc.VectorSubcoreMesh(core_axis_name="c", subcore_axis_name="s")
    k = pl.kernel(
        _sc_body,
        out_type=jax.ShapeDtypeStruct((_B,), jnp.float32),
        mesh=mesh,
        scratch_types=[
            pltpu.VMEM((_NIDX,), jnp.int32),
            pltpu.VMEM((_NIDX,), jnp.int32),
            pltpu.VMEM((_NIDX,), jnp.float32),
            pltpu.VMEM((_ND * _NB,), jnp.float32),
            pltpu.VMEM(((_ND + 1) * _L,), jnp.float32),
            pltpu.VMEM((_NB,), jnp.float32),
            pltpu.SemaphoreType.DMA((_NSTREAM,)),
        ],
    )
    return k(cat_prep, off_flat, num_prep, cst, table_flat)


def kernel(numbers, bias, lin_w, cat_params, categories, embed_idx):
    cat_prep = (
        categories.reshape(_NW, _NB, _F).transpose(0, 2, 1).reshape(_NW, _NIDX)
    )
    off_flat = jnp.repeat(embed_idx.astype(jnp.int32), _NB)
    num_prep = (
        numbers.reshape(_NW, _NB, _ND).transpose(0, 2, 1).reshape(_NW, _ND * _NB)
    )
    cst = jnp.concatenate(
        [jnp.repeat(lin_w.reshape(_ND), _L), jnp.repeat(bias.reshape(1), _L)]
    )
    table_flat = cat_params.reshape(-1)
    out = _cat_linear_sc(cat_prep, off_flat, num_prep, cst, table_flat)
    return out.reshape(_B, 1)
